# Initial kernel scaffold; baseline (speedup 1.0000x reference)
import jax
import jax.numpy as jnp
from jax.experimental import pallas as pl


def kernel(x, edge_index, W1l, W1r, b1, W2l, W2r, b2, W3l, W3r, b3):
    raise NotImplementedError("write your pallas kernel here")


# speedup vs baseline: 5.9508x; 5.9508x over previous
import functools

import jax
import jax.numpy as jnp
from jax import lax
from jax.experimental import pallas as pl
from jax.experimental.pallas import tpu as pltpu
from jax.experimental.pallas import tpu_sc as plsc

N, E, D = 10000, 320000, 128
DH = D // 2

NC, NS = 2, 16
CH = 128
CHUNKS = -(-E // (NS * CH))
EPT = CHUNKS * CH
EP = NS * EPT
NP = 10112
RPT = NP // NS


def _sc_agg(with_deg):
    mesh = plsc.VectorSubcoreMesh(core_axis_name="c", subcore_axis_name="s")
    out_type = [jax.ShapeDtypeStruct((NC, NP, DH), jnp.float32)]
    scratch = [
        pltpu.VMEM((CHUNKS, CH), jnp.int32),
        pltpu.VMEM((CHUNKS, CH), jnp.int32),
        pltpu.VMEM((CH, DH), jnp.float32),
        pltpu.VMEM_SHARED((NP, DH), jnp.float32),
        pltpu.SemaphoreType.DMA,
    ]
    if with_deg:
        out_type.append(jax.ShapeDtypeStruct((NP, 16), jnp.float32))
        scratch += [
            pltpu.VMEM((CH, 16), jnp.float32),
            pltpu.VMEM_SHARED((NP, 16), jnp.float32),
        ]

    def body(h, srcr, dstr, zrows, zdeg, ones, agg_out, *rest):
        if with_deg:
            deg_out, src_v, dst_v, rows_v, acc_sh, sem, ones_v, deg_sh = rest
        else:
            src_v, dst_v, rows_v, acc_sh, sem = rest
        c = lax.axis_index("c")
        s = lax.axis_index("s")
        pltpu.sync_copy(srcr.at[s], src_v)
        pltpu.sync_copy(dstr.at[s], dst_v)
        row0 = s * RPT
        pltpu.sync_copy(zrows, acc_sh.at[pl.ds(row0, RPT)])
        if with_deg:
            pltpu.sync_copy(ones, ones_v)

            @pl.when(c == 0)
            def _():
                pltpu.sync_copy(zdeg, deg_sh.at[pl.ds(row0, RPT)])
        plsc.subcore_barrier()

        def chunk(j, carry):
            pltpu.async_copy(h.at[c].at[src_v.at[j]], rows_v, sem).wait()
            pltpu.sync_copy(rows_v, acc_sh.at[dst_v.at[j]], add=True)
            if with_deg:
                @pl.when(c == 0)
                def _():
                    pltpu.sync_copy(ones_v, deg_sh.at[dst_v.at[j]], add=True)
            return carry

        lax.fori_loop(0, CHUNKS, chunk, 0)
        plsc.subcore_barrier()
        pltpu.sync_copy(acc_sh.at[pl.ds(row0, RPT)],
                        agg_out.at[c, pl.ds(row0, RPT)])
        if with_deg:
            @pl.when(c == 0)
            def _():
                pltpu.sync_copy(deg_sh.at[pl.ds(row0, RPT)],
                                deg_out.at[pl.ds(row0, RPT)])

    return pl.kernel(body, out_type, mesh=mesh, scratch_types=scratch,
                     compiler_params=pltpu.CompilerParams(
                         use_tc_tiling_on_sc=False))


_sc_agg_deg_call = _sc_agg(True)
_sc_agg_call = _sc_agg(False)

_BR = 2000


def _tc_body(agg, deg, h, wl, wr, b, out, *, final):
    ssum = jnp.concatenate([agg[0], agg[1]], axis=-1)
    dg = deg[:, 0:1]
    mean = ssum / jnp.maximum(dg, 1.0)
    hcat = jnp.concatenate([h[0], h[1]], axis=-1)
    o = (jnp.dot(mean, wl[...], preferred_element_type=jnp.float32)
         + jnp.dot(hcat, wr[...], preferred_element_type=jnp.float32)
         + b[...])
    if final:
        m = jnp.max(o, axis=-1, keepdims=True)
        lo = o - m
        out[...] = lo - jnp.log(jnp.sum(jnp.exp(lo), axis=-1, keepdims=True))
    else:
        a = jnp.where(o > 0, o, jnp.exp(jnp.minimum(o, 0.0)) - 1.0)
        out[0] = a[:, :DH]
        out[1] = a[:, DH:]


def _tc_layer(agg, deg, h, wl, wr, b, final):
    if final:
        out_spec = pl.BlockSpec((_BR, D), lambda i: (i, 0))
        out_shape = jax.ShapeDtypeStruct((N, D), jnp.float32)
    else:
        out_spec = pl.BlockSpec((NC, _BR, DH), lambda i: (0, i, 0))
        out_shape = jax.ShapeDtypeStruct((NC, N, DH), jnp.float32)
    return pl.pallas_call(
        functools.partial(_tc_body, final=final),
        grid=(N // _BR,),
        in_specs=[
            pl.BlockSpec((NC, _BR, DH), lambda i: (0, i, 0)),
            pl.BlockSpec((_BR, 16), lambda i: (i, 0)),
            pl.BlockSpec((NC, _BR, DH), lambda i: (0, i, 0)),
            pl.BlockSpec((D, D), lambda i: (0, 0)),
            pl.BlockSpec((D, D), lambda i: (0, 0)),
            pl.BlockSpec((1, D), lambda i: (0, 0)),
        ],
        out_specs=out_spec,
        out_shape=out_shape,
    )(agg, deg, h, wl, wr, b)


def kernel(x, edge_index, W1l, W1r, b1, W2l, W2r, b2, W3l, W3r, b3):
    pad = EP - E
    src = jnp.concatenate([edge_index[0], jnp.zeros((pad,), jnp.int32)])
    dst = jnp.concatenate([edge_index[1], jnp.full((pad,), N, jnp.int32)])
    src_r = src.reshape(NS, CHUNKS, CH)
    dst_r = dst.reshape(NS, CHUNKS, CH)
    zrows = jnp.zeros((RPT, DH), jnp.float32)
    zdeg = jnp.zeros((RPT, 16), jnp.float32)
    ones = jnp.ones((CH, 16), jnp.float32)
    xs = x.reshape(N, NC, DH).transpose(1, 0, 2)

    agg1, deg = _sc_agg_deg_call(xs, src_r, dst_r, zrows, zdeg, ones)
    h1 = _tc_layer(agg1, deg, xs, W1l, W1r, b1.reshape(1, D), final=False)
    agg2, = _sc_agg_call(h1, src_r, dst_r, zrows, zdeg, ones)
    h2 = _tc_layer(agg2, deg, h1, W2l, W2r, b2.reshape(1, D), final=False)
    agg3, = _sc_agg_call(h2, src_r, dst_r, zrows, zdeg, ones)
    return _tc_layer(agg3, deg, h2, W3l, W3r, b3.reshape(1, D), final=True)
